# baseline (device time: 87148 ns/iter reference)
import jax
import jax.numpy as jnp
from jax import lax
from jax.experimental import pallas as pl
from jax.experimental.pallas import tpu as pltpu

N_DEV = 4


def kernel(A, B):
    m, _ = A.shape
    _, n = B.shape
    chunk = m // N_DEV

    def body(a_ref, b_ref, out_ref, comm_ref, send_sems, recv_sems):
        my = lax.axis_index("i")
        left = lax.rem(my + N_DEV - 1, N_DEV)
        right = lax.rem(my + 1, N_DEV)

        barrier_sem = pltpu.get_barrier_semaphore()
        for nbr in (left, right):
            pl.semaphore_signal(
                barrier_sem, inc=1,
                device_id=(nbr,), device_id_type=pl.DeviceIdType.MESH,
            )
        pl.semaphore_wait(barrier_sem, 2)

        out_ref[:, :] = jnp.dot(
            a_ref[:, :], b_ref[:, :], preferred_element_type=jnp.float32
        )

        for s in range(N_DEV - 1):
            sc = lax.rem(my - s + N_DEV, N_DEV)
            rdma = pltpu.make_async_remote_copy(
                src_ref=out_ref.at[pl.ds(sc * chunk, chunk), :],
                dst_ref=comm_ref.at[s],
                send_sem=send_sems.at[s],
                recv_sem=recv_sems.at[s],
                device_id=(right,),
                device_id_type=pl.DeviceIdType.MESH,
            )
            rdma.start()
            rdma.wait()
            rc = lax.rem(my - s - 1 + N_DEV, N_DEV)
            rows = pl.ds(rc * chunk, chunk)
            out_ref[rows, :] = out_ref[rows, :] + comm_ref[s]

        own = lax.rem(my + 1, N_DEV)
        own_rows = pl.ds(own * chunk, chunk)
        out_ref[own_rows, :] = jnp.maximum(out_ref[own_rows, :], 0.0)

        for s in range(N_DEV - 1):
            g = (N_DEV - 1) + s
            sc = lax.rem(my + 1 - s + N_DEV, N_DEV)
            rows = pl.ds(sc * chunk, chunk)
            rdma = pltpu.make_async_remote_copy(
                src_ref=out_ref.at[rows, :],
                dst_ref=out_ref.at[rows, :],
                send_sem=send_sems.at[g],
                recv_sem=recv_sems.at[g],
                device_id=(right,),
                device_id_type=pl.DeviceIdType.MESH,
            )
            rdma.start()
            rdma.wait()

    return pl.pallas_call(
        body,
        out_shape=jax.ShapeDtypeStruct((m, n), jnp.float32),
        in_specs=[
            pl.BlockSpec(memory_space=pltpu.VMEM),
            pl.BlockSpec(memory_space=pltpu.VMEM),
        ],
        out_specs=pl.BlockSpec(memory_space=pltpu.VMEM),
        scratch_shapes=[
            pltpu.VMEM((N_DEV - 1, chunk, n), jnp.float32),
            pltpu.SemaphoreType.DMA((2 * (N_DEV - 1),)),
            pltpu.SemaphoreType.DMA((2 * (N_DEV - 1),)),
        ],
        compiler_params=pltpu.CompilerParams(collective_id=0),
    )(A, B)


# device time: 53744 ns/iter; 1.6215x vs baseline; 1.6215x over previous
import jax
import jax.numpy as jnp
from jax import lax
from jax.experimental import pallas as pl
from jax.experimental.pallas import tpu as pltpu

N_DEV = 4


def kernel(A, B):
    m, _ = A.shape
    _, n = B.shape
    chunk = m // N_DEV

    def body(a_ref, b_ref, out_ref,
             rs_stage, rs_comm, ag_stage, ag_comm, send_sems, recv_sems):
        my = lax.axis_index("i")
        left = lax.rem(my + N_DEV - 1, N_DEV)
        right = lax.rem(my + 1, N_DEV)

        barrier_sem = pltpu.get_barrier_semaphore()
        for nbr in (left, right):
            pl.semaphore_signal(
                barrier_sem, inc=1,
                device_id=(nbr,), device_id_type=pl.DeviceIdType.MESH,
            )
        pl.semaphore_wait(barrier_sem, 2)

        out_ref[:, :] = jnp.dot(
            a_ref[:, :], b_ref[:, :], preferred_element_type=jnp.float32
        )

        rs_stage[0] = out_ref[pl.ds(my * chunk, chunk), :].astype(jnp.bfloat16)

        for s in range(N_DEV - 1):
            rdma = pltpu.make_async_remote_copy(
                src_ref=rs_stage.at[s],
                dst_ref=rs_comm.at[s],
                send_sem=send_sems.at[s],
                recv_sem=recv_sems.at[s],
                device_id=(right,),
                device_id_type=pl.DeviceIdType.MESH,
            )
            rdma.start()
            rdma.wait()
            rc = lax.rem(my - s - 1 + N_DEV, N_DEV)
            rows = pl.ds(rc * chunk, chunk)
            if s < N_DEV - 2:
                rs_stage[s + 1] = (
                    rs_comm[s] + out_ref[rows, :].astype(jnp.bfloat16)
                )
            else:
                red = jnp.maximum(
                    out_ref[rows, :] + rs_comm[s].astype(jnp.float32), 0.0
                )
                out_ref[rows, :] = red
                ag_stage[:, :] = red.astype(jnp.bfloat16)

        for s in range(N_DEV - 1):
            g = (N_DEV - 1) + s
            src = ag_stage if s == 0 else ag_comm.at[s - 1]
            rdma = pltpu.make_async_remote_copy(
                src_ref=src,
                dst_ref=ag_comm.at[s],
                send_sem=send_sems.at[g],
                recv_sem=recv_sems.at[g],
                device_id=(right,),
                device_id_type=pl.DeviceIdType.MESH,
            )
            rdma.start()
            rdma.wait()
            rc = lax.rem(my - s + N_DEV, N_DEV)
            rows = pl.ds(rc * chunk, chunk)
            out_ref[rows, :] = ag_comm[s].astype(jnp.float32)

    return pl.pallas_call(
        body,
        out_shape=jax.ShapeDtypeStruct((m, n), jnp.float32),
        in_specs=[
            pl.BlockSpec(memory_space=pltpu.VMEM),
            pl.BlockSpec(memory_space=pltpu.VMEM),
        ],
        out_specs=pl.BlockSpec(memory_space=pltpu.VMEM),
        scratch_shapes=[
            pltpu.VMEM((N_DEV - 1, chunk, n), jnp.bfloat16),
            pltpu.VMEM((N_DEV - 1, chunk, n), jnp.bfloat16),
            pltpu.VMEM((chunk, n), jnp.bfloat16),
            pltpu.VMEM((N_DEV - 1, chunk, n), jnp.bfloat16),
            pltpu.SemaphoreType.DMA((2 * (N_DEV - 1),)),
            pltpu.SemaphoreType.DMA((2 * (N_DEV - 1),)),
        ],
        compiler_params=pltpu.CompilerParams(collective_id=0),
    )(A, B)


# device time: 35954 ns/iter; 2.4239x vs baseline; 1.4948x over previous
import jax
import jax.numpy as jnp
from jax import lax
from jax.experimental import pallas as pl
from jax.experimental.pallas import tpu as pltpu

N_DEV = 4


def kernel(A, B):
    m, _ = A.shape
    _, n = B.shape
    chunk = m // N_DEV
    half = chunk // 2

    def body(a_ref, b_ref, out_ref,
             stage_r, comm_r, stage_l, comm_l,
             ag_stage_r, ag_comm_r, ag_stage_l, ag_comm_l,
             send_r, recv_r, send_l, recv_l):
        my = lax.axis_index("i")
        left = lax.rem(my + N_DEV - 1, N_DEV)
        right = lax.rem(my + 1, N_DEV)

        barrier_sem = pltpu.get_barrier_semaphore()
        for nbr in (left, right):
            pl.semaphore_signal(
                barrier_sem, inc=1,
                device_id=(nbr,), device_id_type=pl.DeviceIdType.MESH,
            )
        pl.semaphore_wait(barrier_sem, 2)

        def compute_block(c):
            rows = pl.ds(c * chunk, chunk)
            out_ref[rows, :] = jnp.dot(
                a_ref[rows, :], b_ref[:, :],
                preferred_element_type=jnp.float32,
            )

        def hop(src_r, src_l, dst_r, dst_l, s_slot, during=None):
            rd_r = pltpu.make_async_remote_copy(
                src_ref=src_r, dst_ref=dst_r,
                send_sem=send_r.at[s_slot], recv_sem=recv_r.at[s_slot],
                device_id=(right,), device_id_type=pl.DeviceIdType.MESH,
            )
            rd_l = pltpu.make_async_remote_copy(
                src_ref=src_l, dst_ref=dst_l,
                send_sem=send_l.at[s_slot], recv_sem=recv_l.at[s_slot],
                device_id=(left,), device_id_type=pl.DeviceIdType.MESH,
            )
            rd_r.start()
            rd_l.start()
            if during is not None:
                during()
            rd_r.wait()
            rd_l.wait()

        compute_block(my)
        top0 = pl.ds(my * chunk, half)
        bot0 = pl.ds(my * chunk + half, half)
        stage_r[0] = out_ref[top0, :].astype(jnp.bfloat16)
        stage_l[0] = out_ref[bot0, :].astype(jnp.bfloat16)

        for s in range(N_DEV - 1):
            during = None
            if s == 0:
                def during():
                    compute_block(lax.rem(my + N_DEV - 1, N_DEV))
                    compute_block(lax.rem(my + 1, N_DEV))
            elif s == 1:
                def during():
                    compute_block(lax.rem(my + 2, N_DEV))
            hop(stage_r.at[s], stage_l.at[s],
                comm_r.at[s], comm_l.at[s], s, during)

            rc_r = lax.rem(my - s - 1 + N_DEV, N_DEV)
            rc_l = lax.rem(my + s + 1, N_DEV)
            rows_rt = pl.ds(rc_r * chunk, half)
            rows_lb = pl.ds(rc_l * chunk + half, half)
            if s < N_DEV - 2:
                stage_r[s + 1] = (
                    comm_r[s] + out_ref[rows_rt, :].astype(jnp.bfloat16)
                )
                stage_l[s + 1] = (
                    comm_l[s] + out_ref[rows_lb, :].astype(jnp.bfloat16)
                )
            else:
                red_t = jnp.maximum(
                    out_ref[rows_rt, :] + comm_r[s].astype(jnp.float32), 0.0
                )
                out_ref[rows_rt, :] = red_t
                ag_stage_r[:, :] = red_t.astype(jnp.bfloat16)
                red_b = jnp.maximum(
                    out_ref[rows_lb, :] + comm_l[s].astype(jnp.float32), 0.0
                )
                out_ref[rows_lb, :] = red_b
                ag_stage_l[:, :] = red_b.astype(jnp.bfloat16)

        for s in range(N_DEV - 1):
            g = (N_DEV - 1) + s
            src_r = ag_stage_r if s == 0 else ag_comm_r.at[s - 1]
            src_l = ag_stage_l if s == 0 else ag_comm_l.at[s - 1]

            def during():
                if s > 0:
                    p = s - 1
                    pr_r = lax.rem(my - p + N_DEV, N_DEV)
                    pr_l = lax.rem(my + p, N_DEV)
                    out_ref[pl.ds(pr_r * chunk, half), :] = (
                        ag_comm_r[p].astype(jnp.float32)
                    )
                    out_ref[pl.ds(pr_l * chunk + half, half), :] = (
                        ag_comm_l[p].astype(jnp.float32)
                    )
            hop(src_r, src_l, ag_comm_r.at[s], ag_comm_l.at[s], g, during)

        p = N_DEV - 2
        pr_r = lax.rem(my - p + N_DEV, N_DEV)
        pr_l = lax.rem(my + p, N_DEV)
        out_ref[pl.ds(pr_r * chunk, half), :] = ag_comm_r[p].astype(jnp.float32)
        out_ref[pl.ds(pr_l * chunk + half, half), :] = (
            ag_comm_l[p].astype(jnp.float32)
        )

    n_hops = 2 * (N_DEV - 1)
    return pl.pallas_call(
        body,
        out_shape=jax.ShapeDtypeStruct((m, n), jnp.float32),
        in_specs=[
            pl.BlockSpec(memory_space=pltpu.VMEM),
            pl.BlockSpec(memory_space=pltpu.VMEM),
        ],
        out_specs=pl.BlockSpec(memory_space=pltpu.VMEM),
        scratch_shapes=[
            pltpu.VMEM((N_DEV - 1, half, n), jnp.bfloat16),
            pltpu.VMEM((N_DEV - 1, half, n), jnp.bfloat16),
            pltpu.VMEM((N_DEV - 1, half, n), jnp.bfloat16),
            pltpu.VMEM((N_DEV - 1, half, n), jnp.bfloat16),
            pltpu.VMEM((half, n), jnp.bfloat16),
            pltpu.VMEM((N_DEV - 1, half, n), jnp.bfloat16),
            pltpu.VMEM((half, n), jnp.bfloat16),
            pltpu.VMEM((N_DEV - 1, half, n), jnp.bfloat16),
            pltpu.SemaphoreType.DMA((n_hops,)),
            pltpu.SemaphoreType.DMA((n_hops,)),
            pltpu.SemaphoreType.DMA((n_hops,)),
            pltpu.SemaphoreType.DMA((n_hops,)),
        ],
        compiler_params=pltpu.CompilerParams(collective_id=0),
    )(A, B)
